# baseline (speedup 1.0000x reference)
import functools

import jax
import jax.numpy as jnp
from jax.experimental import pallas as pl
from jax.experimental.pallas import tpu as pltpu

N_TOK = 16384
D_MODEL = 4096
N_EXP = 64
N_GROUPS = 8
GROUP_SIZE = N_EXP // N_GROUPS
TOPK_GROUPS = 4
TOPK = 8
ROUTE_SCALE = 2.5

BLK_TOK = 1024

_NEG = float("-inf")


def _gate_kernel(x_ref, w_ref, b_ref, wout_ref, iout_ref):
    blk = x_ref.shape[0]
    logits = jax.lax.dot_general(
        w_ref[...], x_ref[...],
        dimension_numbers=(((1,), (1,)), ((), ())),
        preferred_element_type=jnp.float32,
    )
    scores = jax.nn.sigmoid(logits)
    sb = scores + b_ref[...]

    x3 = sb.reshape(N_GROUPS, GROUP_SIZE, blk)
    i1 = jax.lax.broadcasted_iota(jnp.int32, (N_GROUPS, GROUP_SIZE, blk), 1)
    m1 = jnp.max(x3, axis=1, keepdims=True)
    f1 = jnp.min(jnp.where(x3 == m1, i1, GROUP_SIZE), axis=1, keepdims=True)
    m2 = jnp.max(jnp.where(i1 == f1, _NEG, x3), axis=1, keepdims=True)
    gs = m1 + m2

    gi = jax.lax.broadcasted_iota(jnp.int32, (N_GROUPS, 1, blk), 0)
    gmask = jnp.zeros((N_GROUPS, 1, blk), dtype=jnp.bool_)
    for _ in range(TOPK_GROUPS):
        gm = jnp.max(gs, axis=0, keepdims=True)
        gf = jnp.min(jnp.where(gs == gm, gi, N_GROUPS), axis=0, keepdims=True)
        sel = gi == gf
        gmask = jnp.logical_or(gmask, sel)
        gs = jnp.where(sel, _NEG, gs)

    allowed = jnp.broadcast_to(gmask, (N_GROUPS, GROUP_SIZE, blk))
    masked = jnp.where(allowed.reshape(N_EXP, blk), sb, _NEG)

    ei = jax.lax.broadcasted_iota(jnp.int32, (N_EXP, blk), 0)
    idx_rows = []
    w_rows = []
    for _ in range(TOPK):
        m = jnp.max(masked, axis=0, keepdims=True)
        f = jnp.min(jnp.where(masked == m, ei, N_EXP), axis=0, keepdims=True)
        sel = ei == f
        w_rows.append(jnp.sum(jnp.where(sel, scores, 0.0), axis=0, keepdims=True))
        idx_rows.append(f)
        masked = jnp.where(sel, _NEG, masked)

    w8 = jnp.concatenate(w_rows, axis=0)
    i8 = jnp.concatenate(idx_rows, axis=0)
    wsum = jnp.sum(w8, axis=0, keepdims=True)
    w8 = w8 / (wsum + 1e-6) * ROUTE_SCALE

    wout_ref[...] = w8.T
    iout_ref[...] = i8.T


@functools.partial(jax.jit, static_argnames=())
def kernel(hidden_states, weight, bias):
    n_tok = hidden_states.shape[0]
    bias2d = bias.reshape(N_EXP, 1).astype(jnp.float32)
    grid = (n_tok // BLK_TOK,)
    wout, iout = pl.pallas_call(
        _gate_kernel,
        grid=grid,
        in_specs=[
            pl.BlockSpec((BLK_TOK, D_MODEL), lambda i: (i, 0)),
            pl.BlockSpec((N_EXP, D_MODEL), lambda i: (0, 0)),
            pl.BlockSpec((N_EXP, 1), lambda i: (0, 0)),
        ],
        out_specs=[
            pl.BlockSpec((BLK_TOK, TOPK), lambda i: (i, 0)),
            pl.BlockSpec((BLK_TOK, TOPK), lambda i: (i, 0)),
        ],
        out_shape=[
            jax.ShapeDtypeStruct((n_tok, TOPK), jnp.float32),
            jax.ShapeDtypeStruct((n_tok, TOPK), jnp.int32),
        ],
        compiler_params=pltpu.CompilerParams(
            dimension_semantics=("parallel",),
        ),
    )(hidden_states, weight, bias2d)
    return wout.astype(hidden_states.dtype), iout


# speedup vs baseline: 1.2101x; 1.2101x over previous
import functools

import jax
import jax.numpy as jnp
from jax.experimental import pallas as pl
from jax.experimental.pallas import tpu as pltpu

N_TOK = 16384
D_MODEL = 4096
N_EXP = 64
N_GROUPS = 8
GROUP_SIZE = N_EXP // N_GROUPS
TOPK_GROUPS = 4
TOPK = 8
ROUTE_SCALE = 2.5

BLK_TOK = 1024

_NEG = float("-inf")


def _gate_kernel(x_ref, w_ref, b_ref, wout_ref, iout_ref):
    blk = x_ref.shape[0]
    logits = jax.lax.dot_general(
        w_ref[...], x_ref[...],
        dimension_numbers=(((1,), (1,)), ((), ())),
        preferred_element_type=jnp.float32,
    )
    scores = jax.nn.sigmoid(logits)
    sb = scores + b_ref[...]

    x3 = sb.reshape(N_GROUPS, GROUP_SIZE, blk)
    i1 = jax.lax.broadcasted_iota(jnp.int32, (N_GROUPS, GROUP_SIZE, blk), 1)
    m1 = jnp.max(x3, axis=1, keepdims=True)
    f1 = jnp.min(jnp.where(x3 == m1, i1, GROUP_SIZE), axis=1, keepdims=True)
    m2 = jnp.max(jnp.where(i1 == f1, _NEG, x3), axis=1, keepdims=True)
    gs = m1 + m2

    gi = jax.lax.broadcasted_iota(jnp.int32, (N_GROUPS, 1, blk), 0)
    gmask = jnp.zeros((N_GROUPS, 1, blk), dtype=jnp.bool_)
    for _ in range(TOPK_GROUPS):
        gm = jnp.max(gs, axis=0, keepdims=True)
        gf = jnp.min(jnp.where(gs == gm, gi, N_GROUPS), axis=0, keepdims=True)
        sel = gi == gf
        gmask = jnp.logical_or(gmask, sel)
        gs = jnp.where(sel, _NEG, gs)

    allowed = jnp.broadcast_to(gmask, (N_GROUPS, GROUP_SIZE, blk))
    masked = jnp.where(allowed.reshape(N_EXP, blk), sb, _NEG)

    ei = jax.lax.broadcasted_iota(jnp.int32, (N_EXP, blk), 0)
    idx_rows = []
    w_rows = []
    for _ in range(TOPK):
        m = jnp.max(masked, axis=0, keepdims=True)
        f = jnp.min(jnp.where(masked == m, ei, N_EXP), axis=0, keepdims=True)
        sel = ei == f
        w_rows.append(jnp.sum(jnp.where(sel, scores, 0.0), axis=0, keepdims=True))
        idx_rows.append(f)
        masked = jnp.where(sel, _NEG, masked)

    w8 = jnp.concatenate(w_rows, axis=0)
    i8 = jnp.concatenate(idx_rows, axis=0)
    wsum = jnp.sum(w8, axis=0, keepdims=True)
    w8 = w8 / (wsum + 1e-6) * ROUTE_SCALE

    wout_ref[...] = w8
    iout_ref[...] = i8


@functools.partial(jax.jit, static_argnames=())
def kernel(hidden_states, weight, bias):
    n_tok = hidden_states.shape[0]
    bias2d = bias.reshape(N_EXP, 1).astype(jnp.float32)
    grid = (n_tok // BLK_TOK,)
    wout, iout = pl.pallas_call(
        _gate_kernel,
        grid=grid,
        in_specs=[
            pl.BlockSpec((BLK_TOK, D_MODEL), lambda i: (i, 0)),
            pl.BlockSpec((N_EXP, D_MODEL), lambda i: (0, 0)),
            pl.BlockSpec((N_EXP, 1), lambda i: (0, 0)),
        ],
        out_specs=[
            pl.BlockSpec((TOPK, BLK_TOK), lambda i: (0, i)),
            pl.BlockSpec((TOPK, BLK_TOK), lambda i: (0, i)),
        ],
        out_shape=[
            jax.ShapeDtypeStruct((TOPK, n_tok), jnp.float32),
            jax.ShapeDtypeStruct((TOPK, n_tok), jnp.int32),
        ],
        compiler_params=pltpu.CompilerParams(
            dimension_semantics=("parallel",),
        ),
    )(hidden_states, weight, bias2d)
    return wout.T.astype(hidden_states.dtype), iout.T
